# Initial kernel scaffold; baseline (speedup 1.0000x reference)
import jax
import jax.numpy as jnp
from jax.experimental import pallas as pl


def kernel(input, pool_weight, import_params, hash_values):
    raise NotImplementedError("write your pallas kernel here")


# speedup vs baseline: 8.1205x; 8.1205x over previous
import functools

import jax
import jax.numpy as jnp
from jax import lax
from jax.experimental import pallas as pl
from jax.experimental.pallas import tpu as pltpu
from jax.experimental.pallas import tpu_sc as plsc

_BATCH, _SEQ, _DIM = 4096, 50, 64
_N = _BATCH * _SEQ
_NW = 32
_TPW = _N // _NW
_CHUNK_SUB = 5
_C = _CHUNK_SUB * 128
_NCHUNK = _TPW // _C
_G = _C // 16


def _sc_embed(ids, pool, hv0_tab, hv1_tab, ip0_tab, ip1_tab):
    mesh = plsc.VectorSubcoreMesh(core_axis_name="c", subcore_axis_name="s")

    @functools.partial(
        pl.kernel,
        mesh=mesh,
        compiler_params=pltpu.CompilerParams(use_tc_tiling_on_sc=False),
        out_type=jax.ShapeDtypeStruct((_N, _DIM), jnp.float32),
        scratch_types=[
            pltpu.VMEM((_C,), jnp.int32),
            pltpu.VMEM((_C,), jnp.int32),
            pltpu.VMEM((_C,), jnp.int32),
            pltpu.VMEM((_C,), jnp.float32),
            pltpu.VMEM((_C,), jnp.float32),
            pltpu.VMEM((_C, _DIM), jnp.float32),
            pltpu.VMEM((_C, _DIM), jnp.float32),
            pltpu.SemaphoreType.DMA,
            pltpu.SemaphoreType.DMA,
        ],
    )
    def k(ids_hbm, pool_hbm, hv0_hbm, hv1_hbm, ip0_hbm, ip1_hbm, out_hbm,
          ids_v, hv0_v, hv1_v, ip0_v, ip1_v, r0_v, r1_v, sem_h, sem_r):
        wid = lax.axis_index("s") * 2 + lax.axis_index("c")
        t0w = wid * _TPW

        def chunk_body(c, carry):
            base = t0w + c * _C
            pltpu.sync_copy(ids_hbm.at[pl.ds(base, _C)], ids_v)

            cps = []
            for j in range(_CHUNK_SUB):
                sl = pl.ds(j * 128, 128)
                idx = ids_v.at[sl]
                cps.append(pltpu.async_copy(hv0_hbm.at[idx], hv0_v.at[sl],
                                            sem_h))
                cps.append(pltpu.async_copy(hv1_hbm.at[idx], hv1_v.at[sl],
                                            sem_h))
                cps.append(pltpu.async_copy(ip0_hbm.at[idx], ip0_v.at[sl],
                                            sem_h))
                cps.append(pltpu.async_copy(ip1_hbm.at[idx], ip1_v.at[sl],
                                            sem_h))
            for cp in cps:
                cp.wait()

            cps = []
            for j in range(_CHUNK_SUB):
                sl = pl.ds(j * 128, 128)
                cps.append(pltpu.async_copy(pool_hbm.at[hv0_v.at[sl]],
                                            r0_v.at[sl], sem_r))
                cps.append(pltpu.async_copy(pool_hbm.at[hv1_v.at[sl]],
                                            r1_v.at[sl], sem_r))
            for cp in cps:
                cp.wait()

            def group_body(g, carry2):
                t0 = g * 16
                wv0 = ip0_v[pl.ds(t0, 16)]
                wv1 = ip1_v[pl.ds(t0, 16)]
                for j in range(16):
                    t = t0 + j
                    w0 = wv0[j]
                    w1 = wv1[j]
                    for q in range(4):
                        a = r0_v[t, pl.ds(q * 16, 16)]
                        b = r1_v[t, pl.ds(q * 16, 16)]
                        r0_v[t, pl.ds(q * 16, 16)] = w0 * a + w1 * b
                return carry2

            lax.fori_loop(0, _G, group_body, 0)
            pltpu.sync_copy(r0_v, out_hbm.at[pl.ds(base, _C)])
            return carry

        lax.fori_loop(0, _NCHUNK, chunk_body, 0)

    return k(ids, pool, hv0_tab, hv1_tab, ip0_tab, ip1_tab)


def kernel(input, pool_weight, import_params, hash_values):
    ids = input.reshape(_N).astype(jnp.int32)
    hv0_tab = hash_values[:, 0].astype(jnp.int32)
    hv1_tab = hash_values[:, 1].astype(jnp.int32)
    ip0_tab = import_params[:, 0] * 1.0
    ip1_tab = import_params[:, 1] * 1.0
    out = _sc_embed(ids, pool_weight, hv0_tab, hv1_tab, ip0_tab, ip1_tab)
    return out.reshape(_BATCH, _SEQ, _DIM)


# speedup vs baseline: 9.4448x; 1.1631x over previous
import functools

import jax
import jax.numpy as jnp
from jax import lax
from jax.experimental import pallas as pl
from jax.experimental.pallas import tpu as pltpu
from jax.experimental.pallas import tpu_sc as plsc

_BATCH, _SEQ, _DIM = 4096, 50, 64
_N = _BATCH * _SEQ
_NW = 32
_TPW = _N // _NW
_CHUNK_SUB = 2
_C = _CHUNK_SUB * 128
_NCHUNK = _TPW // _C
_G = _C // 16


def _sc_embed(ids, pool, hv0_tab, hv1_tab, ip0_tab, ip1_tab):
    mesh = plsc.VectorSubcoreMesh(core_axis_name="c", subcore_axis_name="s")

    @functools.partial(
        pl.kernel,
        mesh=mesh,
        compiler_params=pltpu.CompilerParams(use_tc_tiling_on_sc=False),
        out_type=jax.ShapeDtypeStruct((_N, _DIM), jnp.float32),
        scratch_types=[
            pltpu.VMEM((_TPW,), jnp.int32),
            [pltpu.VMEM((_C,), jnp.int32) for _ in range(3)],
            [pltpu.VMEM((_C,), jnp.int32) for _ in range(3)],
            [pltpu.VMEM((_C,), jnp.float32) for _ in range(3)],
            [pltpu.VMEM((_C,), jnp.float32) for _ in range(3)],
            [pltpu.VMEM((_C, _DIM), jnp.float32) for _ in range(2)],
            [pltpu.VMEM((_C, _DIM), jnp.float32) for _ in range(2)],
            [pltpu.SemaphoreType.DMA for _ in range(3)],
            [pltpu.SemaphoreType.DMA for _ in range(2)],
            [pltpu.SemaphoreType.DMA for _ in range(2)],
        ],
    )
    def k(ids_hbm, pool_hbm, hv0_hbm, hv1_hbm, ip0_hbm, ip1_hbm, out_hbm,
          ids_v, hv0_v, hv1_v, ip0_v, ip1_v, r0_v, r1_v,
          sem_side, sem_rows, sem_out):
        wid = lax.axis_index("s") * 2 + lax.axis_index("c")
        t0w = wid * _TPW

        pltpu.sync_copy(ids_hbm.at[pl.ds(t0w, _TPW)], ids_v)

        def issue_side(n):
            p = n % 3
            idx = ids_v.at[pl.ds(n * _C, _C)]
            return [
                pltpu.async_copy(hv0_hbm.at[idx], hv0_v[p], sem_side[p]),
                pltpu.async_copy(hv1_hbm.at[idx], hv1_v[p], sem_side[p]),
                pltpu.async_copy(ip0_hbm.at[idx], ip0_v[p], sem_side[p]),
                pltpu.async_copy(ip1_hbm.at[idx], ip1_v[p], sem_side[p]),
            ]

        def issue_rows(n):
            p3 = n % 3
            p = n % 2
            return [
                pltpu.async_copy(pool_hbm.at[hv0_v[p3]], r0_v[p], sem_rows[p]),
                pltpu.async_copy(pool_hbm.at[hv1_v[p3]], r1_v[p], sem_rows[p]),
            ]

        def compute(n):
            p = n % 2
            p3 = n % 3
            r0p, r1p, ip0p, ip1p = r0_v[p], r1_v[p], ip0_v[p3], ip1_v[p3]

            def group_body(g, carry):
                t0 = g * 16
                wv0 = ip0p[pl.ds(t0, 16)]
                wv1 = ip1p[pl.ds(t0, 16)]
                for j in range(16):
                    t = t0 + j
                    w0 = wv0[j]
                    w1 = wv1[j]
                    for q in range(4):
                        a = r0p[t, pl.ds(q * 16, 16)]
                        b = r1p[t, pl.ds(q * 16, 16)]
                        r0p[t, pl.ds(q * 16, 16)] = w0 * a + w1 * b
                return carry

            lax.fori_loop(0, _G, group_body, 0)

        side_cps = {0: issue_side(0), 1: issue_side(1)}
        for cp in side_cps[0]:
            cp.wait()
        rows_cps = {0: issue_rows(0)}
        out_cps = {}

        for c in range(_NCHUNK):
            p = c % 2
            pn = (c + 1) % 2
            if c + 2 < _NCHUNK:
                side_cps[c + 2] = issue_side(c + 2)
            if c + 1 < _NCHUNK:
                for cp in side_cps.pop(c + 1):
                    cp.wait()
                if c >= 1:
                    for cp in out_cps.pop(c - 1):
                        cp.wait()
                rows_cps[c + 1] = issue_rows(c + 1)
            for cp in rows_cps.pop(c):
                cp.wait()
            compute(c)
            out_cps[c] = [pltpu.async_copy(
                r0_v[p], out_hbm.at[pl.ds(t0w + c * _C, _C)], sem_out[p])]

        for c in sorted(out_cps):
            for cp in out_cps[c]:
                cp.wait()

    return k(ids, pool, hv0_tab, hv1_tab, ip0_tab, ip1_tab)


def kernel(input, pool_weight, import_params, hash_values):
    ids = input.reshape(_N).astype(jnp.int32)
    hv0_tab = hash_values[:, 0].astype(jnp.int32)
    hv1_tab = hash_values[:, 1].astype(jnp.int32)
    ip0_tab = import_params[:, 0] * 1.0
    ip1_tab = import_params[:, 1] * 1.0
    out = _sc_embed(ids, pool_weight, hv0_tab, hv1_tab, ip0_tab, ip1_tab)
    return out.reshape(_BATCH, _SEQ, _DIM)
